# baseline (speedup 1.0000x reference)
import functools

import numpy as np
import jax
import jax.numpy as jnp
from jax import lax
from jax.experimental import pallas as pl
from jax.experimental.pallas import tpu as pltpu
from jax.experimental.pallas import tpu_sc as plsc

TOPK = 5
MAX_CANDIDATES = 32
MAX_LAG_CAP = 168

PADB = 256
T = 2048
T1 = 1024


def _mean_kernel(x_ref, out_ref):
    out_ref[0, 0, :] = jnp.sum(x_ref[0], axis=-1) * (1.0 / x_ref.shape[2])


def _score_topk_kernel(lags_arr, m_ref, w_ref):
    B, L = m_ref.shape
    num = lags_arr.shape[0]
    m = m_ref[...]
    cols = []
    for j in range(num):
        lag = int(lags_arr[j])
        prod = m[:, : L - lag] * m[:, lag:]
        cols.append(jnp.sum(prod, axis=1, keepdims=True) * (1.0 / (L - lag)))
    scores = jnp.concatenate(cols, axis=1)

    iota = lax.broadcasted_iota(jnp.int32, (B, num), 1)
    neg_big = jnp.float32(-3.0e38)
    vals = []
    sel_idx = []
    s = scores
    for _ in range(TOPK):
        mx = jnp.max(s, axis=1, keepdims=True)
        idx = jnp.min(jnp.where(s == mx, iota, num), axis=1, keepdims=True)
        sel = iota == idx
        vals.append(mx)
        sel_idx.append(idx)
        s = jnp.where(sel, neg_big, s)
    vals = jnp.concatenate(vals, axis=1)
    idxs = jnp.concatenate(sel_idx, axis=1)
    denom = jnp.sum(vals, axis=1, keepdims=True) + 1e-6
    w = vals / denom
    w_full = jnp.zeros((B, num), jnp.float32)
    for k in range(TOPK):
        w_full = w_full + jnp.where(iota == idxs[:, k : k + 1],
                                    w[:, k : k + 1], 0.0)
    w_ref[...] = w_full


def _sc_scores_body(lags, L, B, m_hbm, part_hbm, mpad, svec):
    num = len(lags)
    npad = mpad.shape[0]
    c = lax.axis_index("c")
    s = lax.axis_index("s")
    b = 2 * c + s // 8
    q = s % 8

    pltpu.sync_copy(m_hbm.at[b], mpad.at[pl.ds(0, L)])
    zeros16 = jnp.zeros((16,), jnp.float32)
    for k in range((npad - L) // 16):
        mpad[pl.ds(L + 16 * k, 16)] = zeros16

    jidx = [q * 4 + k for k in range(4)]
    lag_k = []
    inv_k = []
    for k in range(4):
        lag = jnp.int32(0)
        inv = jnp.float32(0.0)
        for j in range(num):
            lag = jnp.where(jidx[k] == j, jnp.int32(int(lags[j])), lag)
            inv = jnp.where(jidx[k] == j, jnp.float32(1.0 / (L - int(lags[j]))), inv)
        lag_k.append(lag)
        inv_k.append(inv)

    def body(i, accs):
        base = i * 16
        v = mpad[pl.ds(base, 16)]
        return tuple(
            accs[k] + v * mpad[pl.ds(base + lag_k[k], 16)] for k in range(4)
        )

    accs = lax.fori_loop(0, L // 16, body, (zeros16,) * 4)

    iota = lax.iota(jnp.int32, 16)
    h0 = jnp.zeros((16,), jnp.float32)
    h1 = jnp.zeros((16,), jnp.float32)
    for k in range(4):
        sc = jnp.sum(accs[k]) * inv_k[k]
        h0 = jnp.where(iota == jidx[k], sc, h0)
        h1 = jnp.where(iota == jidx[k] - 16, sc, h1)
    svec[pl.ds(0, 16)] = h0
    svec[pl.ds(16, 16)] = h1
    pltpu.sync_copy(svec, part_hbm.at[b, q])


def _sc_topk_body(B, part_hbm, w_hbm, gbuf, svec):
    c = lax.axis_index("c")
    s = lax.axis_index("s")
    b = s * 2 + c

    @pl.when(b < B)
    def _():
        pltpu.sync_copy(part_hbm.at[b], gbuf)
        iota = lax.iota(jnp.int32, 16)
        s0 = jnp.zeros((16,), jnp.float32)
        s1 = jnp.zeros((16,), jnp.float32)
        for qq in range(8):
            s0 = s0 + gbuf[qq, pl.ds(0, 16)]
            s1 = s1 + gbuf[qq, pl.ds(16, 16)]

        neg_big = jnp.float32(-3.0e38)
        vals = []
        idxs = []
        for _ in range(TOPK):
            mx = jnp.maximum(jnp.max(s0), jnp.max(s1))
            i0 = jnp.min(jnp.where(s0 == mx, iota, jnp.int32(99)))
            i1 = jnp.min(jnp.where(s1 == mx, iota + 16, jnp.int32(99)))
            idx = jnp.minimum(i0, i1)
            vals.append(mx)
            idxs.append(idx)
            s0 = jnp.where(iota == idx, neg_big, s0)
            s1 = jnp.where(iota + 16 == idx, neg_big, s1)
        denom = vals[0] + vals[1] + vals[2] + vals[3] + vals[4] + jnp.float32(1e-6)
        v0 = jnp.zeros((16,), jnp.float32)
        v1 = jnp.zeros((16,), jnp.float32)
        for k in range(TOPK):
            v0 = jnp.where(iota == idxs[k], vals[k], v0)
            v1 = jnp.where(iota + 16 == idxs[k], vals[k], v1)
        dv = jnp.full((16,), denom, jnp.float32)
        svec[pl.ds(0, 16)] = v0 / dv
        svec[pl.ds(16, 16)] = v1 / dv
        pltpu.sync_copy(svec, w_hbm.at[b])


def _agg_kernel(lags_arr, w_ref, prev_ref, cur_ref, out_ref):
    b = pl.program_id(0)
    out_ref[0] = jnp.zeros_like(out_ref)[0]
    for j in range(lags_arr.shape[0]):
        lag = int(lags_arr[j])
        wj = w_ref[b, j]

        @pl.when(wj != 0.0)
        def _(lag=lag, wj=wj):
            head = prev_ref[0, PADB - lag : PADB, :]
            tail = cur_ref[0, 0 : T - lag, :]
            out_ref[0] += wj * jnp.concatenate([head, tail], axis=0)


@jax.jit
def kernel(x):
    B, L, D = x.shape
    assert L % T == 0 and L % PADB == 0 and T % PADB == 0

    max_lag = min(L - 1, MAX_LAG_CAP)
    num = min(max_lag, MAX_CANDIDATES)
    lags_np = np.linspace(1.0, float(max_lag), num=num).astype(np.int64)

    x_mean = pl.pallas_call(
        _mean_kernel,
        grid=(B, L // T1),
        in_specs=[pl.BlockSpec((1, T1, D), lambda b, t: (b, t, 0))],
        out_specs=pl.BlockSpec((1, 1, T1), lambda b, t: (b, 0, t)),
        out_shape=jax.ShapeDtypeStruct((B, 1, L), jnp.float32),
    )(x)
    x_mean = x_mean.reshape(B, L)

    npad = L + ((MAX_LAG_CAP + 31) // 16) * 16
    sc_mesh = plsc.VectorSubcoreMesh(core_axis_name="c", subcore_axis_name="s")
    sc_params = pltpu.CompilerParams(needs_layout_passes=False)
    parts = pl.kernel(
        functools.partial(_sc_scores_body, [int(v) for v in lags_np], L, B),
        out_type=jax.ShapeDtypeStruct((B, 8, num), jnp.float32),
        mesh=sc_mesh,
        compiler_params=sc_params,
        scratch_types=[
            pltpu.VMEM((npad,), jnp.float32),
            pltpu.VMEM((num,), jnp.float32),
        ],
    )(x_mean)
    w32 = pl.kernel(
        functools.partial(_sc_topk_body, B),
        out_type=jax.ShapeDtypeStruct((B, num), jnp.float32),
        mesh=sc_mesh,
        compiler_params=sc_params,
        scratch_types=[
            pltpu.VMEM((8, num), jnp.float32),
            pltpu.VMEM((num,), jnp.float32),
        ],
    )(parts)

    NPB = L // PADB
    R = T // PADB
    out = pl.pallas_call(
        functools.partial(_agg_kernel, lags_np),
        grid=(B, L // T),
        in_specs=[
            pl.BlockSpec(memory_space=pltpu.SMEM),
            pl.BlockSpec((1, PADB, D), lambda b, t: (b, (t * R - 1) % NPB, 0)),
            pl.BlockSpec((1, T, D), lambda b, t: (b, t, 0)),
        ],
        out_specs=pl.BlockSpec((1, T, D), lambda b, t: (b, t, 0)),
        out_shape=jax.ShapeDtypeStruct((B, L, D), jnp.float32),
    )(w32, x, x)
    return out


# speedup vs baseline: 2.6119x; 2.6119x over previous
import functools

import numpy as np
import jax
import jax.numpy as jnp
from jax import lax
from jax.experimental import pallas as pl
from jax.experimental.pallas import tpu as pltpu
from jax.experimental.pallas import tpu_sc as plsc

TOPK = 5
MAX_CANDIDATES = 32
MAX_LAG_CAP = 168

PADB = 256
T = 1024
T1 = 1024


def _mean_kernel(x_ref, out_ref):
    out_ref[0, 0, :] = jnp.sum(x_ref[0], axis=-1) * (1.0 / x_ref.shape[2])


def _score_topk_kernel(lags_arr, m_ref, w_ref):
    B, L = m_ref.shape
    num = lags_arr.shape[0]
    m = m_ref[...]
    cols = []
    for j in range(num):
        lag = int(lags_arr[j])
        prod = m[:, : L - lag] * m[:, lag:]
        cols.append(jnp.sum(prod, axis=1, keepdims=True) * (1.0 / (L - lag)))
    scores = jnp.concatenate(cols, axis=1)

    iota = lax.broadcasted_iota(jnp.int32, (B, num), 1)
    neg_big = jnp.float32(-3.0e38)
    vals = []
    sel_idx = []
    s = scores
    for _ in range(TOPK):
        mx = jnp.max(s, axis=1, keepdims=True)
        idx = jnp.min(jnp.where(s == mx, iota, num), axis=1, keepdims=True)
        sel = iota == idx
        vals.append(mx)
        sel_idx.append(idx)
        s = jnp.where(sel, neg_big, s)
    vals = jnp.concatenate(vals, axis=1)
    idxs = jnp.concatenate(sel_idx, axis=1)
    denom = jnp.sum(vals, axis=1, keepdims=True) + 1e-6
    w = vals / denom
    w_full = jnp.zeros((B, num), jnp.float32)
    for k in range(TOPK):
        w_full = w_full + jnp.where(iota == idxs[:, k : k + 1],
                                    w[:, k : k + 1], 0.0)
    w_ref[...] = w_full


def _sc_scores_body(lags, L, B, m_hbm, part_hbm, mpad, svec):
    num = len(lags)
    npad = mpad.shape[0]
    c = lax.axis_index("c")
    s = lax.axis_index("s")
    b = 2 * c + s // 8
    q = s % 8

    pltpu.sync_copy(m_hbm.at[b], mpad.at[pl.ds(0, L)])
    zeros16 = jnp.zeros((16,), jnp.float32)
    for k in range((npad - L) // 16):
        mpad[pl.ds(L + 16 * k, 16)] = zeros16

    jidx = [q * 4 + k for k in range(4)]
    lag_k = []
    inv_k = []
    for k in range(4):
        lag = jnp.int32(0)
        inv = jnp.float32(0.0)
        for j in range(num):
            lag = jnp.where(jidx[k] == j, jnp.int32(int(lags[j])), lag)
            inv = jnp.where(jidx[k] == j, jnp.float32(1.0 / (L - int(lags[j]))), inv)
        lag_k.append(lag)
        inv_k.append(inv)

    def body(i, accs):
        base = i * 16
        v = mpad[pl.ds(base, 16)]
        return tuple(
            accs[k] + v * mpad[pl.ds(base + lag_k[k], 16)] for k in range(4)
        )

    accs = lax.fori_loop(0, L // 16, body, (zeros16,) * 4)

    iota = lax.iota(jnp.int32, 16)
    h0 = jnp.zeros((16,), jnp.float32)
    h1 = jnp.zeros((16,), jnp.float32)
    for k in range(4):
        sc = jnp.sum(accs[k]) * inv_k[k]
        h0 = jnp.where(iota == jidx[k], sc, h0)
        h1 = jnp.where(iota == jidx[k] - 16, sc, h1)
    svec[pl.ds(0, 16)] = h0
    svec[pl.ds(16, 16)] = h1
    pltpu.sync_copy(svec, part_hbm.at[b, q])


def _sc_topk_body(B, part_hbm, w_hbm, gbuf, svec):
    c = lax.axis_index("c")
    s = lax.axis_index("s")
    b = s * 2 + c

    @pl.when(b < B)
    def _():
        pltpu.sync_copy(part_hbm.at[b], gbuf)
        iota = lax.iota(jnp.int32, 16)
        s0 = jnp.zeros((16,), jnp.float32)
        s1 = jnp.zeros((16,), jnp.float32)
        for qq in range(8):
            s0 = s0 + gbuf[qq, pl.ds(0, 16)]
            s1 = s1 + gbuf[qq, pl.ds(16, 16)]

        neg_big = jnp.float32(-3.0e38)
        vals = []
        idxs = []
        for _ in range(TOPK):
            mx = jnp.maximum(jnp.max(s0), jnp.max(s1))
            i0 = jnp.min(jnp.where(s0 == mx, iota, jnp.int32(99)))
            i1 = jnp.min(jnp.where(s1 == mx, iota + 16, jnp.int32(99)))
            idx = jnp.minimum(i0, i1)
            vals.append(mx)
            idxs.append(idx)
            s0 = jnp.where(iota == idx, neg_big, s0)
            s1 = jnp.where(iota + 16 == idx, neg_big, s1)
        denom = vals[0] + vals[1] + vals[2] + vals[3] + vals[4] + jnp.float32(1e-6)
        v0 = jnp.zeros((16,), jnp.float32)
        v1 = jnp.zeros((16,), jnp.float32)
        for k in range(TOPK):
            v0 = jnp.where(iota == idxs[k], vals[k], v0)
            v1 = jnp.where(iota + 16 == idxs[k], vals[k], v1)
        dv = jnp.full((16,), denom, jnp.float32)
        svec[pl.ds(0, 16)] = v0 / dv
        svec[pl.ds(16, 16)] = v1 / dv
        pltpu.sync_copy(svec, w_hbm.at[b])


def _agg_kernel(lags_arr, w_ref, prev_ref, cur_ref, out_ref):
    b = pl.program_id(0)
    out_ref[0] = jnp.zeros_like(out_ref)[0]
    for j in range(lags_arr.shape[0]):
        lag = int(lags_arr[j])
        wj = w_ref[b, j]

        @pl.when(wj != 0.0)
        def _(lag=lag, wj=wj):
            head = prev_ref[0, PADB - lag : PADB, :]
            tail = cur_ref[0, 0 : T - lag, :]
            out_ref[0] += wj * jnp.concatenate([head, tail], axis=0)


@jax.jit
def kernel(x):
    B, L, D = x.shape
    assert L % T == 0 and L % PADB == 0 and T % PADB == 0

    max_lag = min(L - 1, MAX_LAG_CAP)
    num = min(max_lag, MAX_CANDIDATES)
    lags_np = np.linspace(1.0, float(max_lag), num=num).astype(np.int64)

    x_mean = pl.pallas_call(
        _mean_kernel,
        grid=(B, L // T1),
        in_specs=[pl.BlockSpec((1, T1, D), lambda b, t: (b, t, 0))],
        out_specs=pl.BlockSpec((1, 1, T1), lambda b, t: (b, 0, t)),
        out_shape=jax.ShapeDtypeStruct((B, 1, L), jnp.float32),
    )(x)
    x_mean = x_mean.reshape(B, L)

    npad = L + ((MAX_LAG_CAP + 31) // 16) * 16
    sc_mesh = plsc.VectorSubcoreMesh(core_axis_name="c", subcore_axis_name="s")
    sc_params = pltpu.CompilerParams(needs_layout_passes=False)
    parts = pl.kernel(
        functools.partial(_sc_scores_body, [int(v) for v in lags_np], L, B),
        out_type=jax.ShapeDtypeStruct((B, 8, num), jnp.float32),
        mesh=sc_mesh,
        compiler_params=sc_params,
        scratch_types=[
            pltpu.VMEM((npad,), jnp.float32),
            pltpu.VMEM((num,), jnp.float32),
        ],
    )(x_mean)
    w32 = pl.kernel(
        functools.partial(_sc_topk_body, B),
        out_type=jax.ShapeDtypeStruct((B, num), jnp.float32),
        mesh=sc_mesh,
        compiler_params=sc_params,
        scratch_types=[
            pltpu.VMEM((8, num), jnp.float32),
            pltpu.VMEM((num,), jnp.float32),
        ],
    )(parts)

    NPB = L // PADB
    R = T // PADB
    out = pl.pallas_call(
        functools.partial(_agg_kernel, lags_np),
        grid=(B, L // T),
        in_specs=[
            pl.BlockSpec(memory_space=pltpu.SMEM),
            pl.BlockSpec((1, PADB, D), lambda b, t: (b, (t * R - 1) % NPB, 0)),
            pl.BlockSpec((1, T, D), lambda b, t: (b, t, 0)),
        ],
        out_specs=pl.BlockSpec((1, T, D), lambda b, t: (b, t, 0)),
        out_shape=jax.ShapeDtypeStruct((B, L, D), jnp.float32),
    )(w32, x, x)
    return out
